# baseline (speedup 1.0000x reference)
import math

import jax
import jax.numpy as jnp
from jax import lax
from jax.experimental import pallas as pl

_SMOOTHING = 0.1
_CONFIDENCE = 1.0 - _SMOOTHING
_PAD = 0
_BLOCK_R = 32


def _tc_body(block_r, v, eps, c1, half_blocks):
    def _rows_term(xv, t):
        cols = lax.broadcasted_iota(jnp.int32, (block_r, v), 1)
        s = jnp.sum(xv, axis=1, keepdims=True)
        g = jnp.sum(jnp.where(cols == t, xv, 0.0), axis=1, keepdims=True)
        x0 = xv[:, 0:1]
        valid = (t != _PAD).astype(jnp.float32)
        per_row = valid * (c1 - eps * s + eps * x0 + (eps - _CONFIDENCE) * g)
        return jnp.sum(per_row, keepdims=True)

    def body(t_lo_ref, t_hi_ref, x_lo_ref, x_hi_ref, out_ref):
        i = pl.program_id(0)
        partial = (_rows_term(x_lo_ref[:, :], t_lo_ref[:, :])
                   + _rows_term(x_hi_ref[:, :], t_hi_ref[:, :]))

        @pl.when(i == 0)
        def _init():
            out_ref[:, :] = jnp.zeros_like(out_ref)

        out_ref[:, :] += partial

    return body


def kernel(x, target):
    batch, v = x.shape
    eps = _SMOOTHING / (v - 2)
    c1 = eps * math.log(eps) * (v - 2) + _CONFIDENCE * math.log(_CONFIDENCE)
    half_blocks = batch // (2 * _BLOCK_R)

    t2 = target.astype(jnp.int32).reshape(batch, 1)

    out = pl.pallas_call(
        _tc_body(_BLOCK_R, v, eps, c1, half_blocks),
        grid=(half_blocks,),
        in_specs=[
            pl.BlockSpec((_BLOCK_R, 1), lambda i: (i, 0)),
            pl.BlockSpec((_BLOCK_R, 1), lambda i, hb=half_blocks: (i + hb, 0)),
            pl.BlockSpec((_BLOCK_R, v), lambda i: (i, 0)),
            pl.BlockSpec((_BLOCK_R, v), lambda i, hb=half_blocks: (i + hb, 0)),
        ],
        out_specs=pl.BlockSpec((1, 1), lambda i: (0, 0)),
        out_shape=jax.ShapeDtypeStruct((1, 1), jnp.float32),
    )(t2, t2, x, x)
    return out[0, 0]


# speedup vs baseline: 1.0151x; 1.0151x over previous
import math

import jax
import jax.numpy as jnp
from jax import lax
from jax.experimental import pallas as pl
from jax.experimental.pallas import tpu as pltpu

_SMOOTHING = 0.1
_CONFIDENCE = 1.0 - _SMOOTHING
_PAD = 0
_BLOCK_R = 32


def _tc_body(block_r, v, eps, c1):
    def body(target_ref, x_ref, out_ref):
        xv = x_ref[:, :]
        t = target_ref[:, :]
        cols = lax.broadcasted_iota(jnp.int32, (block_r, v), 1)
        s = jnp.sum(xv, axis=1, keepdims=True)
        g = jnp.sum(jnp.where(cols == t, xv, 0.0), axis=1, keepdims=True)
        x0 = xv[:, 0:1]
        valid = (t != _PAD).astype(jnp.float32)
        per_row = valid * (c1 - eps * s + eps * x0 + (eps - _CONFIDENCE) * g)
        out_ref[:, :, :] = jnp.sum(per_row, keepdims=True).reshape(1, 1, 1)

    return body


def kernel(x, target):
    batch, v = x.shape
    eps = _SMOOTHING / (v - 2)
    c1 = eps * math.log(eps) * (v - 2) + _CONFIDENCE * math.log(_CONFIDENCE)
    nblocks = batch // _BLOCK_R

    t2 = target.astype(jnp.int32).reshape(batch, 1)

    partials = pl.pallas_call(
        _tc_body(_BLOCK_R, v, eps, c1),
        grid=(nblocks,),
        in_specs=[
            pl.BlockSpec((_BLOCK_R, 1), lambda i: (i, 0)),
            pl.BlockSpec((_BLOCK_R, v), lambda i: (i, 0)),
        ],
        out_specs=pl.BlockSpec((1, 1, 1), lambda i: (i, 0, 0)),
        out_shape=jax.ShapeDtypeStruct((nblocks, 1, 1), jnp.float32),
        compiler_params=pltpu.CompilerParams(
            dimension_semantics=("parallel",)),
    )(t2, x)
    return jnp.sum(partials)
